# Initial kernel scaffold; baseline (speedup 1.0000x reference)
import jax
import jax.numpy as jnp
from jax.experimental import pallas as pl


def kernel(x, edge_index, W0, W1, W2):
    raise NotImplementedError("write your pallas kernel here")


# speedup vs baseline: 4.4651x; 4.4651x over previous
import functools

import jax
import jax.numpy as jnp
from jax import lax
from jax.experimental import pallas as pl
from jax.experimental.pallas import tpu as pltpu
from jax.experimental.pallas import tpu_sc as plsc

N_NODES = 10000
N_EDGES = 320000

NC = 2
NS = 16
NW = NC * NS
LANES = 16

EDGES_PER_TILE = N_EDGES // NW
CHUNK = 80
ITERS = EDGES_PER_TILE // CHUNK
ROWS_PER_TILE = 624
EXTRA_ROWS = N_NODES - NS * ROWS_PER_TILE
ZROWS = 104


def _sc_aggregate(m, src, dst):
    F = m.shape[1]
    mesh = plsc.VectorSubcoreMesh(
        core_axis_name="c", subcore_axis_name="s", num_cores=NC, num_subcores=NS
    )

    @functools.partial(
        pl.kernel,
        out_type=jax.ShapeDtypeStruct((NC, N_NODES, F), jnp.float32),
        mesh=mesh,
        scratch_types=[
            pltpu.VMEM((CHUNK,), jnp.int32),
            pltpu.VMEM((CHUNK,), jnp.int32),
            pltpu.VMEM((CHUNK, F), jnp.float32),
            pltpu.VMEM((ZROWS, F), jnp.float32),
            pltpu.VMEM_SHARED((N_NODES, F), jnp.float32),
            pltpu.SemaphoreType.DMA,
        ],
    )
    def k(m_hbm, src_hbm, dst_hbm, out_hbm, src_v, dst_v, rows_v, zero_v, acc_sh, sem):
        cid = lax.axis_index("c")
        sid = lax.axis_index("s")
        wid = sid * NC + cid

        def zrow(i, _):
            def zcol(j, _):
                zero_v[i, pl.ds(j * LANES, LANES)] = jnp.zeros((LANES,), jnp.float32)
                return 0
            lax.fori_loop(0, F // LANES, zcol, 0)
            return 0

        lax.fori_loop(0, ZROWS, zrow, 0)

        rbase = sid * ROWS_PER_TILE
        def zcopy(i, _):
            pltpu.sync_copy(zero_v, acc_sh.at[pl.ds(rbase + i * ZROWS, ZROWS)])
            return 0
        lax.fori_loop(0, ROWS_PER_TILE // ZROWS, zcopy, 0)

        @pl.when(sid == 0)
        def _zero_tail():
            pltpu.sync_copy(
                zero_v.at[pl.ds(0, EXTRA_ROWS)],
                acc_sh.at[pl.ds(NS * ROWS_PER_TILE, EXTRA_ROWS)],
            )

        plsc.subcore_barrier()

        ebase = wid * EDGES_PER_TILE

        def body(j, _):
            off = ebase + j * CHUNK
            pltpu.sync_copy(src_hbm.at[pl.ds(off, CHUNK)], src_v)
            pltpu.sync_copy(dst_hbm.at[pl.ds(off, CHUNK)], dst_v)
            pltpu.async_copy(m_hbm.at[src_v], rows_v, sem).wait()
            pltpu.sync_copy(rows_v, acc_sh.at[dst_v], add=True)
            return 0

        lax.fori_loop(0, ITERS, body, 0)

        plsc.subcore_barrier()

        pltpu.sync_copy(
            acc_sh.at[pl.ds(rbase, ROWS_PER_TILE)],
            out_hbm.at[cid, pl.ds(rbase, ROWS_PER_TILE)],
        )

        @pl.when(sid == 0)
        def _write_tail():
            pltpu.sync_copy(
                acc_sh.at[pl.ds(NS * ROWS_PER_TILE, EXTRA_ROWS)],
                out_hbm.at[cid, pl.ds(NS * ROWS_PER_TILE, EXTRA_ROWS)],
            )

    return k(m, src, dst)


def _tc_matmul(x, W):
    def body(x_ref, w_ref, o_ref):
        o_ref[...] = jnp.dot(
            x_ref[...], w_ref[...],
            preferred_element_type=jnp.float32,
            precision=lax.Precision.HIGHEST,
        )

    return pl.pallas_call(
        body,
        out_shape=jax.ShapeDtypeStruct((x.shape[0], W.shape[1]), jnp.float32),
    )(x, W)


def _tc_combine_relu_matmul(p, W):
    def body(p_ref, w_ref, o_ref):
        h = jnp.maximum(p_ref[0] + p_ref[1], 0.0)
        o_ref[...] = jnp.dot(
            h, w_ref[...],
            preferred_element_type=jnp.float32,
            precision=lax.Precision.HIGHEST,
        )

    return pl.pallas_call(
        body,
        out_shape=jax.ShapeDtypeStruct((p.shape[1], W.shape[1]), jnp.float32),
    )(p, W)


def _tc_combine_relu(p):
    def body(p_ref, o_ref):
        o_ref[...] = jnp.maximum(p_ref[0] + p_ref[1], 0.0)

    return pl.pallas_call(
        body,
        out_shape=jax.ShapeDtypeStruct(p.shape[1:], jnp.float32),
    )(p)


def _tc_combine_matmul(p, W):
    def body(p_ref, w_ref, o_ref):
        o_ref[...] = jnp.dot(
            p_ref[0] + p_ref[1], w_ref[...],
            preferred_element_type=jnp.float32,
            precision=lax.Precision.HIGHEST,
        )

    return pl.pallas_call(
        body,
        out_shape=jax.ShapeDtypeStruct((p.shape[1], W.shape[1]), jnp.float32),
    )(p, W)


def kernel(x, edge_index, W0, W1, W2):
    src = edge_index[0].astype(jnp.int32)
    dst = edge_index[1].astype(jnp.int32)

    m0 = _tc_matmul(x, W0)
    p0 = _sc_aggregate(m0, src, dst)
    m1 = _tc_combine_relu_matmul(p0, W1)
    p1 = _sc_aggregate(m1, src, dst)
    h2 = _tc_combine_relu(p1)
    p2 = _sc_aggregate(h2, src, dst)
    return _tc_combine_matmul(p2, W2)


# speedup vs baseline: 13.1088x; 2.9358x over previous
import functools

import jax
import jax.numpy as jnp
from jax import lax
from jax.experimental import pallas as pl
from jax.experimental.pallas import tpu as pltpu
from jax.experimental.pallas import tpu_sc as plsc

N_NODES = 10000
N_EDGES = 320000

NC = 2
NS = 16
NW = NC * NS
LANES = 16

EDGES_PER_TILE = N_EDGES // NW
CHUNK = 64
CHUNKS = EDGES_PER_TILE // CHUNK
TAIL = EDGES_PER_TILE - CHUNKS * CHUNK
NBUF = 4
ROWS_PER_TILE = 624
EXTRA_ROWS = N_NODES - NS * ROWS_PER_TILE


def _sc_aggregate(m, src, dst):
    F = m.shape[1]
    mesh = plsc.VectorSubcoreMesh(
        core_axis_name="c", subcore_axis_name="s", num_cores=NC, num_subcores=NS
    )

    gbytes = CHUNK * F * 4
    dbytes = CHUNK * 4

    @functools.partial(
        pl.kernel,
        out_type=jax.ShapeDtypeStruct((NC, N_NODES, F), jnp.float32),
        mesh=mesh,
        scratch_types=[
            pltpu.VMEM((EDGES_PER_TILE,), jnp.int32),
            [pltpu.VMEM((CHUNK, F), jnp.float32)] * NBUF,
            [pltpu.VMEM((CHUNK,), jnp.int32)] * NBUF,
            pltpu.VMEM((TAIL,), jnp.int32),
            pltpu.VMEM_SHARED((N_NODES, F), jnp.float32),
            pltpu.SemaphoreType.DMA,
            pltpu.SemaphoreType.DMA,
            pltpu.SemaphoreType.DMA,
        ],
    )
    def k(m_hbm, src_hbm, dst_hbm, out_hbm, src_all, rowbufs, dstbufs,
          dtail, acc_sh, gsem, ssem, dsem):
        cid = lax.axis_index("c")
        sid = lax.axis_index("s")
        wid = sid * NC + cid
        ebase = wid * EDGES_PER_TILE

        pltpu.sync_copy(src_hbm.at[pl.ds(ebase, EDGES_PER_TILE)], src_all)
        for b in range(NBUF - 1):
            pltpu.async_copy(
                m_hbm.at[src_all.at[pl.ds(b * CHUNK, CHUNK)]], rowbufs[b], gsem
            )
            pltpu.async_copy(
                dst_hbm.at[pl.ds(ebase + b * CHUNK, CHUNK)], dstbufs[b], dsem
            )

        zbuf = rowbufs[NBUF - 1]

        def zrow(i, _):
            def zcol(j, _):
                zbuf[i, pl.ds(j * LANES, LANES)] = jnp.zeros((LANES,), jnp.float32)
                return 0
            lax.fori_loop(0, F // LANES, zcol, 0)
            return 0

        lax.fori_loop(0, CHUNK, zrow, 0)

        rbase = sid * ROWS_PER_TILE
        nz = ROWS_PER_TILE // CHUNK
        rz = ROWS_PER_TILE - nz * CHUNK

        def zcopy(i, _):
            pltpu.sync_copy(zbuf, acc_sh.at[pl.ds(rbase + i * CHUNK, CHUNK)])
            return 0
        lax.fori_loop(0, nz, zcopy, 0)
        pltpu.sync_copy(
            zbuf.at[pl.ds(0, rz)],
            acc_sh.at[pl.ds(rbase + nz * CHUNK, rz)],
        )

        @pl.when(sid == 0)
        def _zero_tail():
            pltpu.sync_copy(
                zbuf.at[pl.ds(0, EXTRA_ROWS)],
                acc_sh.at[pl.ds(NS * ROWS_PER_TILE, EXTRA_ROWS)],
            )

        plsc.subcore_barrier()

        def wait_rows(sem, buf):
            pltpu.make_async_copy(m_hbm.at[pl.ds(0, CHUNK)], buf, sem).wait()

        def wait_idx(sem, buf):
            pltpu.make_async_copy(dst_hbm.at[pl.ds(0, CHUNK)], buf, sem).wait()

        @pl.loop(0, CHUNKS // NBUF)
        def _grp(o):
            j0 = o * NBUF
            for b in range(NBUF):
                j = j0 + b
                nb = (b + NBUF - 1) % NBUF
                wait_rows(gsem, rowbufs[b])
                if b == 0:
                    @pl.when(j >= 1)
                    def _retire():
                        wait_rows(ssem, rowbufs[nb])
                else:
                    wait_rows(ssem, rowbufs[nb])

                @pl.when(j + NBUF - 1 < CHUNKS)
                def _fire():
                    jn = j + NBUF - 1
                    pltpu.async_copy(
                        m_hbm.at[src_all.at[pl.ds(jn * CHUNK, CHUNK)]],
                        rowbufs[nb], gsem,
                    )
                    pltpu.async_copy(
                        dst_hbm.at[pl.ds(ebase + jn * CHUNK, CHUNK)],
                        dstbufs[nb], dsem,
                    )

                wait_idx(dsem, dstbufs[b])
                pltpu.async_copy(rowbufs[b], acc_sh.at[dstbufs[b]], ssem, add=True)

        wait_rows(ssem, rowbufs[(CHUNKS - 1) % NBUF])

        tb = ebase + CHUNKS * CHUNK
        pltpu.sync_copy(dst_hbm.at[pl.ds(tb, TAIL)], dtail)
        pltpu.async_copy(
            m_hbm.at[src_all.at[pl.ds(CHUNKS * CHUNK, TAIL)]],
            rowbufs[0].at[pl.ds(0, TAIL)], gsem,
        ).wait()
        pltpu.sync_copy(rowbufs[0].at[pl.ds(0, TAIL)], acc_sh.at[dtail], add=True)

        plsc.subcore_barrier()

        pltpu.sync_copy(
            acc_sh.at[pl.ds(rbase, ROWS_PER_TILE)],
            out_hbm.at[cid, pl.ds(rbase, ROWS_PER_TILE)],
        )

        @pl.when(sid == 0)
        def _write_tail():
            pltpu.sync_copy(
                acc_sh.at[pl.ds(NS * ROWS_PER_TILE, EXTRA_ROWS)],
                out_hbm.at[cid, pl.ds(NS * ROWS_PER_TILE, EXTRA_ROWS)],
            )

    return k(m, src, dst)


def _tc_matmul(x, W):
    def body(x_ref, w_ref, o_ref):
        o_ref[...] = jnp.dot(
            x_ref[...], w_ref[...],
            preferred_element_type=jnp.float32,
            precision=lax.Precision.HIGHEST,
        )

    return pl.pallas_call(
        body,
        out_shape=jax.ShapeDtypeStruct((x.shape[0], W.shape[1]), jnp.float32),
    )(x, W)


def _tc_combine_relu_matmul(p, W):
    def body(p_ref, w_ref, o_ref):
        h = jnp.maximum(p_ref[0] + p_ref[1], 0.0)
        o_ref[...] = jnp.dot(
            h, w_ref[...],
            preferred_element_type=jnp.float32,
            precision=lax.Precision.HIGHEST,
        )

    return pl.pallas_call(
        body,
        out_shape=jax.ShapeDtypeStruct((p.shape[1], W.shape[1]), jnp.float32),
    )(p, W)


def _tc_combine_relu(p):
    def body(p_ref, o_ref):
        o_ref[...] = jnp.maximum(p_ref[0] + p_ref[1], 0.0)

    return pl.pallas_call(
        body,
        out_shape=jax.ShapeDtypeStruct(p.shape[1:], jnp.float32),
    )(p)


def _tc_combine_matmul(p, W):
    def body(p_ref, w_ref, o_ref):
        o_ref[...] = jnp.dot(
            p_ref[0] + p_ref[1], w_ref[...],
            preferred_element_type=jnp.float32,
            precision=lax.Precision.HIGHEST,
        )

    return pl.pallas_call(
        body,
        out_shape=jax.ShapeDtypeStruct((p.shape[1], W.shape[1]), jnp.float32),
    )(p, W)


def kernel(x, edge_index, W0, W1, W2):
    src = edge_index[0].astype(jnp.int32)
    dst = edge_index[1].astype(jnp.int32)

    m0 = _tc_matmul(x, W0)
    p0 = _sc_aggregate(m0, src, dst)
    m1 = _tc_combine_relu_matmul(p0, W1)
    p1 = _sc_aggregate(m1, src, dst)
    h2 = _tc_combine_relu(p1)
    p2 = _sc_aggregate(h2, src, dst)
    return _tc_combine_matmul(p2, W2)
